# baseline (speedup 1.0000x reference)
import dataclasses
import functools

import numpy as np

import jax
import jax.numpy as jnp
from jax.experimental import pallas as pl
from jax.experimental.pallas import tpu as pltpu
from jax.experimental.pallas import tpu_sc as plsc

_NC = 2
_NS = 16
_NW = _NC * _NS


def _linear_body_f32(x_ref, w_ref, b_ref, o_ref):
    o_ref[...] = jax.lax.dot_general(
        x_ref[...], w_ref[...],
        dimension_numbers=(((1,), (1,)), ((), ())),
        preferred_element_type=jnp.float32,
    ) + b_ref[...]


def _linear_body_bf16(x_ref, w_ref, b_ref, o_ref):
    y = jax.lax.dot_general(
        x_ref[...], w_ref[...],
        dimension_numbers=(((1,), (1,)), ((), ())),
        preferred_element_type=jnp.float32,
    ) + b_ref[...]
    o_ref[...] = y.astype(jnp.bfloat16)


def _linear(x, w, b2d, blk, body, out_dtype):
    m, k = x.shape
    dout = w.shape[0]
    return pl.pallas_call(
        body,
        grid=(m // blk,),
        in_specs=[
            pl.BlockSpec((blk, k), lambda i: (i, 0)),
            pl.BlockSpec((dout, k), lambda i: (0, 0)),
            pl.BlockSpec((1, dout), lambda i: (0, 0)),
        ],
        out_specs=pl.BlockSpec((blk, dout), lambda i: (i, 0)),
        out_shape=jax.ShapeDtypeStruct((m, dout), out_dtype),
    )(x, w, b2d)


def _combine_body(a_ref, b_ref, o_ref):
    o_ref[...] = jnp.maximum(a_ref[...] + b_ref[...], 0.0)


def _make_edge_kernel(n_nodes, n_edges, d):
    e_per_tile = n_edges // _NW
    eb = 40
    nblk = e_per_tile // eb
    pub_tiles = 10
    rows_per_pub = n_nodes // pub_tiles
    dw = d // 2

    mesh = plsc.VectorSubcoreMesh(core_axis_name="c", subcore_axis_name="s")

    cp = pltpu.CompilerParams()
    if "needs_layout_passes" in pltpu.CompilerParams.__dataclass_fields__:
        cp = dataclasses.replace(cp, needs_layout_passes=False)
    if "use_tc_tiling_on_sc" in pltpu.CompilerParams.__dataclass_fields__:
        cp = dataclasses.replace(cp, use_tc_tiling_on_sc=False)

    @functools.partial(
        pl.kernel,
        out_type=jax.ShapeDtypeStruct((_NC * n_nodes, d), jnp.float32),
        compiler_params=cp,
        mesh=mesh,
        scratch_types=[
            pltpu.VMEM((e_per_tile,), jnp.int32),
            pltpu.VMEM((e_per_tile,), jnp.int32),
            pltpu.VMEM((eb, d), jnp.float32),
            pltpu.VMEM((eb, d), jnp.float32),
            pltpu.VMEM((eb, dw), jnp.int32),
            pltpu.VMEM((eb, dw), jnp.int32),
            pltpu.VMEM((eb, dw), jnp.int32),
            pltpu.VMEM((eb, dw), jnp.int32),
            pltpu.VMEM_SHARED((n_nodes, d), jnp.float32),
            pltpu.SemaphoreType.DMA,
            pltpu.SemaphoreType.DMA,
            pltpu.SemaphoreType.DMA,
            pltpu.SemaphoreType.DMA,
            pltpu.SemaphoreType.DMA,
            pltpu.SemaphoreType.DMA,
            pltpu.SemaphoreType.DMA,
            pltpu.SemaphoreType.DMA,
        ],
    )
    def edge_kernel(p_hbm, r_hbm, src_hbm, dst_hbm, out_hbm,
                    srcv, dstv, rv0, rv1, ps0, ps1, pd0, pd1, acc,
                    psem0, psem1, dsem0, dsem1, rsem0, rsem1,
                    ssem0, ssem1):
        c = jax.lax.axis_index("c")
        s = jax.lax.axis_index("s")
        wid = c * _NS + s
        base = wid * e_per_tile

        bufs = ((rv0, ps0, pd0, psem0, dsem0, rsem0, ssem0),
                (rv1, ps1, pd1, psem1, dsem1, rsem1, ssem1))

        @pl.when(s < pub_tiles)
        def _():
            @pl.loop(0, eb)
            def _(i):
                for j in range(0, d, 16):
                    rv0[i, pl.ds(j, 16)] = jnp.zeros((16,), jnp.float32)

            @pl.loop(0, rows_per_pub, step=eb)
            def _(k):
                pltpu.sync_copy(rv0, acc.at[pl.ds(s * rows_per_pub + k, eb)])

        plsc.subcore_barrier()

        pltpu.sync_copy(src_hbm.at[pl.ds(base, e_per_tile)], srcv)
        pltpu.sync_copy(dst_hbm.at[pl.ds(base, e_per_tile)], dstv)

        def fetch(g, par):
            rv, ps, pd, psem, dsem, rsem, _ = bufs[par]
            pltpu.async_copy(
                p_hbm.at[srcv.at[pl.ds(g * eb, eb)]], ps, psem)
            pltpu.async_copy(
                p_hbm.at[dstv.at[pl.ds(g * eb, eb)]], pd, dsem)
            pltpu.async_copy(r_hbm.at[pl.ds(base + g * eb, eb)], rv, rsem)

        def wait_fetch(g, par):
            rv, ps, pd, psem, dsem, rsem, _ = bufs[par]
            pltpu.make_async_copy(
                p_hbm.at[srcv.at[pl.ds(g * eb, eb)]], ps, psem).wait()
            pltpu.make_async_copy(
                p_hbm.at[dstv.at[pl.ds(g * eb, eb)]], pd, dsem).wait()
            pltpu.make_async_copy(
                r_hbm.at[pl.ds(base + g * eb, eb)], rv, rsem).wait()

        hi_mask = jnp.full((16,), -65536, jnp.int32)

        fetch(0, 0)

        @pl.loop(0, nblk, step=2)
        def _(g0):
            for par in (0, 1):
                g = g0 + par
                rv, ps, pd, _, _, _, ssem = bufs[par]
                orv, _, _, _, _, _, ossem = bufs[1 - par]

                @pl.when(g < nblk)
                def _():
                    @pl.when(g >= 1)
                    def _():
                        pltpu.make_async_copy(
                            orv, acc.at[dstv.at[pl.ds((g - 1) * eb, eb)]],
                            ossem).wait()

                    @pl.when(g + 1 < nblk)
                    def _():
                        fetch(g + 1, 1 - par)

                    wait_fetch(g, par)

                    @pl.loop(0, eb)
                    def _(e):
                        for jj in range(4):
                            wsl = pl.ds(jj * 16, 16)
                            wps = ps[e, wsl]
                            wpd = pd[e, wsl]
                            ps_lo = plsc.bitcast(
                                jax.lax.shift_left(wps, 16), jnp.float32)
                            ps_hi = plsc.bitcast(
                                jax.lax.bitwise_and(wps, hi_mask),
                                jnp.float32)
                            pd_lo = plsc.bitcast(
                                jax.lax.shift_left(wpd, 16), jnp.float32)
                            pd_hi = plsc.bitcast(
                                jax.lax.bitwise_and(wpd, hi_mask),
                                jnp.float32)
                            lsl = pl.ds(jj * 32, 16)
                            hsl = pl.ds(jj * 32 + 16, 16)
                            rv[e, lsl] = jnp.maximum(
                                rv[e, lsl] + ps_lo + pd_lo, 0.0)
                            rv[e, hsl] = jnp.maximum(
                                rv[e, hsl] + ps_hi + pd_hi, 0.0)

                    pltpu.async_copy(
                        rv, acc.at[dstv.at[pl.ds(g * eb, eb)]], ssem,
                        add=True)

        last = nblk - 1
        lrv = bufs[last % 2][0]
        lsem = bufs[last % 2][6]
        pltpu.make_async_copy(
            lrv, acc.at[dstv.at[pl.ds(last * eb, eb)]], lsem).wait()

        plsc.subcore_barrier()

        @pl.when(s < pub_tiles)
        def _():
            pltpu.sync_copy(
                acc.at[pl.ds(s * rows_per_pub, rows_per_pub)],
                out_hbm.at[pl.ds(c * n_nodes + s * rows_per_pub,
                                 rows_per_pub)],
            )

    return edge_kernel


def kernel(obj_vecs, rel_vecs, edge_index, W_obj, b_obj, W_rel, b_rel):
    n_nodes, d = obj_vecs.shape
    n_edges = rel_vecs.shape[0]

    src = edge_index[:, 0].astype(jnp.int32)
    dst = edge_index[:, 1].astype(jnp.int32)

    perm = np.arange(d).reshape(d // 32, 2, 16).transpose(
        0, 2, 1).reshape(d)
    p_bf = _linear(obj_vecs, W_obj[perm], b_obj[perm].reshape(1, -1), 2000,
                   _linear_body_bf16, jnp.bfloat16)
    p32 = jax.lax.bitcast_convert_type(
        p_bf.reshape(n_nodes, d // 2, 2), jnp.int32)
    r = _linear(rel_vecs, W_rel, b_rel.reshape(1, -1), 2560,
                _linear_body_f32, jnp.float32)

    partials = _make_edge_kernel(n_nodes, n_edges, d)(p32, r, src, dst)

    blk = 2000
    out = pl.pallas_call(
        _combine_body,
        grid=(n_nodes // blk,),
        in_specs=[
            pl.BlockSpec((blk, d), lambda i: (i, 0)),
            pl.BlockSpec((blk, d), lambda i: (i + n_nodes // blk, 0)),
        ],
        out_specs=pl.BlockSpec((blk, d), lambda i: (i, 0)),
        out_shape=jax.ShapeDtypeStruct((n_nodes, d), jnp.float32),
    )(partials, partials)
    return out


# speedup vs baseline: 1.1479x; 1.1479x over previous
import functools

import jax
import jax.numpy as jnp
from jax.experimental import pallas as pl
from jax.experimental.pallas import tpu as pltpu
from jax.experimental.pallas import tpu_sc as plsc

_NC = 2
_NS = 16
_NW = _NC * _NS


def _linear_body(x_ref, w_ref, b_ref, o_ref):
    o_ref[...] = jax.lax.dot_general(
        x_ref[...], w_ref[...],
        dimension_numbers=(((1,), (1,)), ((), ())),
        preferred_element_type=jnp.float32,
    ) + b_ref[...]


def _linear(x, w, b2d, blk):
    m, k = x.shape
    dout = w.shape[0]
    return pl.pallas_call(
        _linear_body,
        grid=(m // blk,),
        in_specs=[
            pl.BlockSpec((blk, k), lambda i: (i, 0)),
            pl.BlockSpec((dout, k), lambda i: (0, 0)),
            pl.BlockSpec((1, dout), lambda i: (0, 0)),
        ],
        out_specs=pl.BlockSpec((blk, dout), lambda i: (i, 0)),
        out_shape=jax.ShapeDtypeStruct((m, dout), jnp.float32),
    )(x, w, b2d)


def _combine_body(a_ref, b_ref, o_ref):
    o_ref[...] = jnp.maximum(a_ref[...] + b_ref[...], 0.0)


def _make_edge_kernel(n_nodes, n_edges, d):
    e_per_tile = n_edges // _NW
    eb = 40
    nblk = e_per_tile // eb
    pub_tiles = 10
    rows_per_pub = n_nodes // pub_tiles

    mesh = plsc.VectorSubcoreMesh(core_axis_name="c", subcore_axis_name="s")

    @functools.partial(
        pl.kernel,
        out_type=jax.ShapeDtypeStruct((_NC * n_nodes, d), jnp.float32),
        mesh=mesh,
        scratch_types=[
            pltpu.VMEM((2 * e_per_tile,), jnp.int32),
            pltpu.VMEM((2 * eb, d), jnp.float32),
            pltpu.VMEM((2 * eb, d), jnp.float32),
            pltpu.VMEM((eb, d), jnp.float32),
            pltpu.VMEM((eb, d), jnp.float32),
            pltpu.VMEM_SHARED((n_nodes, d), jnp.float32),
            pltpu.SemaphoreType.DMA,
            pltpu.SemaphoreType.DMA,
            pltpu.SemaphoreType.DMA,
            pltpu.SemaphoreType.DMA,
            pltpu.SemaphoreType.DMA,
            pltpu.SemaphoreType.DMA,
        ],
    )
    def edge_kernel(p_hbm, r_hbm, cat_hbm, out_hbm,
                    catv, gv0, gv1, rv0, rv1, acc,
                    gsem0, gsem1, rsem0, rsem1, ssem0, ssem1):
        c = jax.lax.axis_index("c")
        s = jax.lax.axis_index("s")
        wid = c * _NS + s
        base = wid * e_per_tile

        bufs = ((gv0, rv0, gsem0, rsem0, ssem0),
                (gv1, rv1, gsem1, rsem1, ssem1))

        @pl.when(s < pub_tiles)
        def _():
            @pl.loop(0, eb)
            def _(i):
                for j in range(0, d, 16):
                    rv0[i, pl.ds(j, 16)] = jnp.zeros((16,), jnp.float32)

            @pl.loop(0, rows_per_pub, step=eb)
            def _(k):
                pltpu.async_copy(
                    rv0, acc.at[pl.ds(s * rows_per_pub + k, eb)], rsem0)

            @pl.loop(0, rows_per_pub, step=eb)
            def _(k):
                pltpu.make_async_copy(
                    rv0, acc.at[pl.ds(s * rows_per_pub + k, eb)],
                    rsem0).wait()

        plsc.subcore_barrier()

        pltpu.sync_copy(cat_hbm.at[pl.ds(2 * base, 2 * e_per_tile)], catv)

        def fetch(g, par):
            gv, rv, gsem, rsem, _ = bufs[par]
            pltpu.async_copy(
                p_hbm.at[catv.at[pl.ds(g * 2 * eb, 2 * eb)]], gv, gsem)
            pltpu.async_copy(r_hbm.at[pl.ds(base + g * eb, eb)], rv, rsem)

        def wait_fetch(g, par):
            gv, rv, gsem, rsem, _ = bufs[par]
            pltpu.make_async_copy(
                p_hbm.at[catv.at[pl.ds(g * 2 * eb, 2 * eb)]], gv,
                gsem).wait()
            pltpu.make_async_copy(
                r_hbm.at[pl.ds(base + g * eb, eb)], rv, rsem).wait()

        fetch(0, 0)

        @pl.loop(0, nblk, step=2)
        def _(g0):
            for par in (0, 1):
                g = g0 + par
                gv, rv, _, _, ssem = bufs[par]
                ogv, orv, _, _, ossem = bufs[1 - par]

                @pl.when(g < nblk)
                def _():
                    @pl.when(g >= 1)
                    def _():
                        pltpu.make_async_copy(
                            orv,
                            acc.at[catv.at[pl.ds((g - 1) * 2 * eb + eb,
                                                 eb)]],
                            ossem).wait()

                    @pl.when(g + 1 < nblk)
                    def _():
                        fetch(g + 1, 1 - par)

                    wait_fetch(g, par)

                    @pl.loop(0, eb)
                    def _(e):
                        for j in range(0, d, 16):
                            sl = pl.ds(j, 16)
                            rv[e, sl] = jnp.maximum(
                                rv[e, sl] + gv[e, sl] + gv[eb + e, sl],
                                0.0)

                    pltpu.async_copy(
                        rv, acc.at[catv.at[pl.ds(g * 2 * eb + eb, eb)]],
                        ssem, add=True)

        last = nblk - 1
        lrv = bufs[last % 2][1]
        lsem = bufs[last % 2][4]
        pltpu.make_async_copy(
            lrv, acc.at[catv.at[pl.ds(last * 2 * eb + eb, eb)]],
            lsem).wait()

        plsc.subcore_barrier()

        @pl.when(s < pub_tiles)
        def _():
            pltpu.sync_copy(
                acc.at[pl.ds(s * rows_per_pub, rows_per_pub)],
                out_hbm.at[pl.ds(c * n_nodes + s * rows_per_pub,
                                 rows_per_pub)],
            )

    return edge_kernel


def kernel(obj_vecs, rel_vecs, edge_index, W_obj, b_obj, W_rel, b_rel):
    n_nodes, d = obj_vecs.shape
    n_edges = rel_vecs.shape[0]
    eb = 40

    src = edge_index[:, 0].astype(jnp.int32).reshape(n_edges // eb, 1, eb)
    dst = edge_index[:, 1].astype(jnp.int32).reshape(n_edges // eb, 1, eb)
    cat = jnp.concatenate([src, dst], axis=1).reshape(-1)

    p = _linear(obj_vecs, W_obj, b_obj.reshape(1, -1), blk=2000)
    r = _linear(rel_vecs, W_rel, b_rel.reshape(1, -1), blk=2560)

    partials = _make_edge_kernel(n_nodes, n_edges, d)(p, r, cat)

    blk = 2000
    out = pl.pallas_call(
        _combine_body,
        grid=(n_nodes // blk,),
        in_specs=[
            pl.BlockSpec((blk, d), lambda i: (i, 0)),
            pl.BlockSpec((blk, d), lambda i: (i + n_nodes // blk, 0)),
        ],
        out_specs=pl.BlockSpec((blk, d), lambda i: (i, 0)),
        out_shape=jax.ShapeDtypeStruct((n_nodes, d), jnp.float32),
    )(partials, partials)
    return out


# speedup vs baseline: 1.3021x; 1.1344x over previous
import functools

import jax
import jax.numpy as jnp
from jax.experimental import pallas as pl
from jax.experimental.pallas import tpu as pltpu
from jax.experimental.pallas import tpu_sc as plsc

_NC = 2
_NS = 16
_NW = _NC * _NS


def _linear_body(x_ref, w_ref, b_ref, o_ref):
    o_ref[...] = jax.lax.dot_general(
        x_ref[...], w_ref[...],
        dimension_numbers=(((1,), (1,)), ((), ())),
        preferred_element_type=jnp.float32,
    ) + b_ref[...]


def _linear(x, w, b2d, blk):
    m, k = x.shape
    dout = w.shape[0]
    return pl.pallas_call(
        _linear_body,
        grid=(m // blk,),
        in_specs=[
            pl.BlockSpec((blk, k), lambda i: (i, 0)),
            pl.BlockSpec((dout, k), lambda i: (0, 0)),
            pl.BlockSpec((1, dout), lambda i: (0, 0)),
        ],
        out_specs=pl.BlockSpec((blk, dout), lambda i: (i, 0)),
        out_shape=jax.ShapeDtypeStruct((m, dout), jnp.float32),
    )(x, w, b2d)


def _combine_body(a_ref, b_ref, o_ref):
    o_ref[...] = jnp.maximum(a_ref[...] + b_ref[...], 0.0)


def _make_edge_kernel(n_nodes, n_edges, d):
    e_per_tile = n_edges // _NW
    eb = 40
    nblk = e_per_tile // eb
    pub_tiles = 10
    rows_per_pub = n_nodes // pub_tiles

    mesh = plsc.VectorSubcoreMesh(core_axis_name="c", subcore_axis_name="s")

    @functools.partial(
        pl.kernel,
        out_type=jax.ShapeDtypeStruct((_NC * n_nodes, d), jnp.float32),
        mesh=mesh,
        scratch_types=[
            pltpu.VMEM((e_per_tile,), jnp.int32),
            pltpu.VMEM((e_per_tile,), jnp.int32),
            pltpu.VMEM((eb, d), jnp.float32),
            pltpu.VMEM((eb, d), jnp.float32),
            pltpu.VMEM((eb, d), jnp.float32),
            pltpu.VMEM((eb, d), jnp.float32),
            pltpu.VMEM((eb, d), jnp.float32),
            pltpu.VMEM((eb, d), jnp.float32),
            pltpu.VMEM_SHARED((n_nodes, d), jnp.float32),
            pltpu.SemaphoreType.DMA,
            pltpu.SemaphoreType.DMA,
            pltpu.SemaphoreType.DMA,
            pltpu.SemaphoreType.DMA,
            pltpu.SemaphoreType.DMA,
            pltpu.SemaphoreType.DMA,
            pltpu.SemaphoreType.DMA,
            pltpu.SemaphoreType.DMA,
        ],
    )
    def edge_kernel(p_hbm, r_hbm, src_hbm, dst_hbm, out_hbm,
                    srcv, dstv, rv0, rv1, ps0, ps1, pd0, pd1, acc,
                    psem0, psem1, dsem0, dsem1, rsem0, rsem1,
                    ssem0, ssem1):
        c = jax.lax.axis_index("c")
        s = jax.lax.axis_index("s")
        wid = c * _NS + s
        base = wid * e_per_tile

        bufs = ((rv0, ps0, pd0, psem0, dsem0, rsem0, ssem0),
                (rv1, ps1, pd1, psem1, dsem1, rsem1, ssem1))

        @pl.when(s < pub_tiles)
        def _():
            @pl.loop(0, eb)
            def _(i):
                for j in range(0, d, 16):
                    rv0[i, pl.ds(j, 16)] = jnp.zeros((16,), jnp.float32)

            @pl.loop(0, rows_per_pub, step=eb)
            def _(k):
                pltpu.async_copy(
                    rv0, acc.at[pl.ds(s * rows_per_pub + k, eb)], rsem0)

            @pl.loop(0, rows_per_pub, step=eb)
            def _(k):
                pltpu.make_async_copy(
                    rv0, acc.at[pl.ds(s * rows_per_pub + k, eb)],
                    rsem0).wait()

        plsc.subcore_barrier()

        pltpu.sync_copy(src_hbm.at[pl.ds(base, e_per_tile)], srcv)
        pltpu.sync_copy(dst_hbm.at[pl.ds(base, e_per_tile)], dstv)

        def fetch(g, par):
            rv, ps, pd, psem, dsem, rsem, _ = bufs[par]
            pltpu.async_copy(
                p_hbm.at[srcv.at[pl.ds(g * eb, eb)]], ps, psem)
            pltpu.async_copy(
                p_hbm.at[dstv.at[pl.ds(g * eb, eb)]], pd, dsem)
            pltpu.async_copy(r_hbm.at[pl.ds(base + g * eb, eb)], rv, rsem)

        def wait_fetch(g, par):
            rv, ps, pd, psem, dsem, rsem, _ = bufs[par]
            pltpu.make_async_copy(
                p_hbm.at[srcv.at[pl.ds(g * eb, eb)]], ps, psem).wait()
            pltpu.make_async_copy(
                p_hbm.at[dstv.at[pl.ds(g * eb, eb)]], pd, dsem).wait()
            pltpu.make_async_copy(
                r_hbm.at[pl.ds(base + g * eb, eb)], rv, rsem).wait()

        fetch(0, 0)

        @pl.loop(0, nblk, step=2)
        def _(g0):
            for par in (0, 1):
                g = g0 + par
                rv, ps, pd, _, _, _, ssem = bufs[par]
                orv, _, _, _, _, _, ossem = bufs[1 - par]

                @pl.when(g < nblk)
                def _():
                    @pl.when(g >= 1)
                    def _():
                        pltpu.make_async_copy(
                            orv, acc.at[dstv.at[pl.ds((g - 1) * eb, eb)]],
                            ossem).wait()

                    @pl.when(g + 1 < nblk)
                    def _():
                        fetch(g + 1, 1 - par)

                    wait_fetch(g, par)

                    @pl.loop(0, eb)
                    def _(e):
                        for j in range(0, d, 16):
                            sl = pl.ds(j, 16)
                            rv[e, sl] = jnp.maximum(
                                rv[e, sl] + ps[e, sl] + pd[e, sl], 0.0)

                    pltpu.async_copy(
                        rv, acc.at[dstv.at[pl.ds(g * eb, eb)]], ssem,
                        add=True)

        last = nblk - 1
        lrv = bufs[last % 2][0]
        lsem = bufs[last % 2][6]
        pltpu.make_async_copy(
            lrv, acc.at[dstv.at[pl.ds(last * eb, eb)]], lsem).wait()

        plsc.subcore_barrier()

        @pl.when(s < pub_tiles)
        def _():
            pltpu.sync_copy(
                acc.at[pl.ds(s * rows_per_pub, rows_per_pub)],
                out_hbm.at[pl.ds(c * n_nodes + s * rows_per_pub,
                                 rows_per_pub)],
            )

    return edge_kernel


def kernel(obj_vecs, rel_vecs, edge_index, W_obj, b_obj, W_rel, b_rel):
    n_nodes, d = obj_vecs.shape
    n_edges = rel_vecs.shape[0]

    src = edge_index[:, 0].astype(jnp.int32)
    dst = edge_index[:, 1].astype(jnp.int32)

    p = _linear(obj_vecs, W_obj, b_obj.reshape(1, -1), blk=2000)
    r = _linear(rel_vecs, W_rel, b_rel.reshape(1, -1), blk=2560)

    partials = _make_edge_kernel(n_nodes, n_edges, d)(p, r, src, dst)

    blk = 2000
    out = pl.pallas_call(
        _combine_body,
        grid=(n_nodes // blk,),
        in_specs=[
            pl.BlockSpec((blk, d), lambda i: (i, 0)),
            pl.BlockSpec((blk, d), lambda i: (i + n_nodes // blk, 0)),
        ],
        out_specs=pl.BlockSpec((blk, d), lambda i: (i, 0)),
        out_shape=jax.ShapeDtypeStruct((n_nodes, d), jnp.float32),
    )(partials, partials)
    return out
